# baseline (speedup 1.0000x reference)
import functools

import jax
import jax.numpy as jnp
from jax import lax
from jax.experimental import pallas as pl
from jax.experimental.pallas import tpu as pltpu
from jax.experimental.pallas import tpu_sc as plsc

_NC = 2
_NS = 16
_NW = _NC * _NS

_K = 256


def _make_matmul_body(BN, K):
    def _matmul_body(x_ref, w_ref, o_ref):
        o_ref[...] = jnp.dot(x_ref[...], w_ref[...],
                             preferred_element_type=jnp.float32)
    return _matmul_body


def _tc_matmul(flat, W2, Nv, K, P):
    BN = 2000
    return pl.pallas_call(
        _make_matmul_body(BN, K),
        grid=(Nv // BN,),
        in_specs=[
            pl.BlockSpec((BN, K), lambda i: (i, 0)),
            pl.BlockSpec((K, P), lambda i: (0, 0)),
        ],
        out_specs=pl.BlockSpec((BN, P), lambda i: (i, 0)),
        out_shape=jax.ShapeDtypeStruct((Nv, P), jnp.float32),
    )(flat, W2)


def _make_scatter(Nv, Bv, OCv):
    nfull = Nv // _K
    part = Nv - nfull * _K
    base_c = nfull // _NW
    extra = nfull - base_c * _NW

    mesh = plsc.VectorSubcoreMesh(core_axis_name="c", subcore_axis_name="s")

    scratch = [
        pltpu.VMEM((Bv, _K), jnp.int32),
        pltpu.VMEM((Bv, _K, OCv), jnp.float32),
        pltpu.SemaphoreType.DMA,
    ]
    if part:
        scratch += [
            pltpu.VMEM((Bv, part), jnp.int32),
            pltpu.VMEM((Bv, part, OCv), jnp.float32),
        ]

    @functools.partial(
        pl.kernel,
        mesh=mesh,
        out_type=(),
        compiler_params=pltpu.CompilerParams(use_tc_tiling_on_sc=False),
        scratch_types=scratch,
    )
    def scatter_kernel(vals_hbm, idx_hbm, out_hbm, idx_v, rows_v, sem,
                       *part_bufs):
        c = lax.axis_index("c")
        s = lax.axis_index("s")
        w = s * _NC + c
        n_w = jnp.where(w < extra, base_c + 1, base_c)
        start_w = w * base_c + jnp.minimum(w, extra)

        def body(i, carry):
            n0 = (start_w + i) * _K
            idx_loads = [
                pltpu.async_copy(idx_hbm.at[pl.ds(j * Nv + n0, _K)],
                                 idx_v.at[j], sem)
                for j in range(Bv)
            ]
            for cp in idx_loads:
                cp.wait()
            loads = [
                pltpu.async_copy(
                    vals_hbm.at[pl.ds(n0, _K), pl.ds(j * OCv, OCv)],
                    rows_v.at[j],
                    sem,
                )
                for j in range(Bv)
            ]
            for cp in loads:
                cp.wait()
            cps = [
                pltpu.async_copy(
                    rows_v.at[j],
                    out_hbm.at[idx_v.at[j]],
                    sem,
                )
                for j in range(Bv)
            ]
            for cp in cps:
                cp.wait()
            return carry

        lax.fori_loop(0, n_w, body, 0)

        if part:
            idx_p, rows_p = part_bufs

            @pl.when(w == extra)
            def _():
                n0 = nfull * _K
                idx_loads = [
                    pltpu.async_copy(idx_hbm.at[pl.ds(j * Nv + n0, part)],
                                     idx_p.at[j], sem)
                    for j in range(Bv)
                ]
                for cp in idx_loads:
                    cp.wait()
                loads = [
                    pltpu.async_copy(
                        vals_hbm.at[pl.ds(n0, part), pl.ds(j * OCv, OCv)],
                        rows_p.at[j],
                        sem,
                    )
                    for j in range(Bv)
                ]
                for cp in loads:
                    cp.wait()
                cps = [
                    pltpu.async_copy(
                        rows_p.at[j],
                        out_hbm.at[idx_p.at[j]],
                        sem,
                    )
                    for j in range(Bv)
                ]
                for cp in cps:
                    cp.wait()

    return scatter_kernel


def _make_zeros(Mv, OCv):
    _ZB = 2048
    per_w = Mv // _NW
    nfull = per_w // _ZB
    part = per_w - nfull * _ZB

    mesh = plsc.VectorSubcoreMesh(core_axis_name="c", subcore_axis_name="s")

    @functools.partial(
        pl.kernel,
        mesh=mesh,
        out_type=jax.ShapeDtypeStruct((Mv, OCv), jnp.float32),
        compiler_params=pltpu.CompilerParams(use_tc_tiling_on_sc=False,
                                             needs_layout_passes=False),
        scratch_types=[
            pltpu.VMEM((_ZB, OCv), jnp.float32),
            pltpu.SemaphoreType.DMA,
        ],
    )
    def zeros_kernel(out_hbm, zb, sem):
        c = lax.axis_index("c")
        s = lax.axis_index("s")
        w = s * _NC + c
        base = w * per_w
        zero16 = jnp.zeros((OCv,), jnp.float32)

        def zfill(g, carry):
            for t in range(16):
                zb[g * 16 + t] = zero16
            return carry

        lax.fori_loop(0, _ZB // 16, zfill, 0)

        def body(i, carry):
            pltpu.sync_copy(zb, out_hbm.at[pl.ds(base + i * _ZB, _ZB)])
            return carry

        lax.fori_loop(0, nfull, body, 0)
        if part:
            pltpu.sync_copy(zb.at[pl.ds(0, part)],
                            out_hbm.at[pl.ds(base + nfull * _ZB, part)])

    return zeros_kernel


def kernel(shuffled_features, mapping, weights):
    Bv, Nv = mapping.shape
    OCv, _, Cv = weights.shape
    Mv = Bv * Nv
    flat = shuffled_features.reshape(Nv, Cv)
    W2 = jnp.transpose(weights, (2, 1, 0)).reshape(Cv, Bv * OCv)
    vals_wide = _tc_matmul(flat, W2, Nv, Cv, Bv * OCv)

    safe = jnp.maximum(mapping, 0).reshape(-1)

    out_ref = jax.new_ref(_make_zeros(Mv, OCv)())
    _make_scatter(Nv, Bv, OCv)(vals_wide, safe, out_ref)
    out = out_ref[...]

    hit = mapping == 0
    has = jnp.any(hit)
    q = jnp.argmax(hit)
    n0 = q % Nv
    j0 = q // Nv
    row0_src = lax.dynamic_slice(vals_wide, (n0, j0 * OCv), (1, OCv))
    row0 = jnp.where(has, row0_src.reshape(OCv),
                     jnp.zeros((OCv,), jnp.float32))
    return out.at[0].set(row0)


# speedup vs baseline: 1.0030x; 1.0030x over previous
import functools

import jax
import jax.numpy as jnp
from jax import lax
from jax.experimental import pallas as pl
from jax.experimental.pallas import tpu as pltpu
from jax.experimental.pallas import tpu_sc as plsc

_NC = 2
_NS = 16
_NW = _NC * _NS

_K = 256


def _make_matmul_body(BN, K):
    def _matmul_body(x_ref, w_ref, o_ref):
        o_ref[...] = jnp.dot(x_ref[...], w_ref[...],
                             preferred_element_type=jnp.float32)
    return _matmul_body


def _tc_matmul(flat, W2, Nv, K, P):
    BN = 2000
    return pl.pallas_call(
        _make_matmul_body(BN, K),
        grid=(Nv // BN,),
        in_specs=[
            pl.BlockSpec((BN, K), lambda i: (i, 0)),
            pl.BlockSpec((K, P), lambda i: (0, 0)),
        ],
        out_specs=pl.BlockSpec((BN, P), lambda i: (i, 0)),
        out_shape=jax.ShapeDtypeStruct((Nv, P), jnp.float32),
    )(flat, W2)


def _make_scatter(Nv, Bv, OCv):
    nfull = Nv // _K
    part = Nv - nfull * _K
    base_c = nfull // _NW
    extra = nfull - base_c * _NW

    mesh = plsc.VectorSubcoreMesh(core_axis_name="c", subcore_axis_name="s")

    scratch = [
        pltpu.VMEM((Bv, _K), jnp.int32),
        pltpu.VMEM((Bv, _K, OCv), jnp.float32),
        pltpu.SemaphoreType.DMA,
    ]
    if part:
        scratch += [
            pltpu.VMEM((Bv, part), jnp.int32),
            pltpu.VMEM((Bv, part, OCv), jnp.float32),
        ]

    @functools.partial(
        pl.kernel,
        mesh=mesh,
        out_type=(),
        compiler_params=pltpu.CompilerParams(use_tc_tiling_on_sc=False),
        scratch_types=scratch,
    )
    def scatter_kernel(vals_hbm, idx_hbm, out_hbm, idx_v, rows_v, sem,
                       *part_bufs):
        c = lax.axis_index("c")
        s = lax.axis_index("s")
        w = s * _NC + c
        n_w = jnp.where(w < extra, base_c + 1, base_c)
        start_w = w * base_c + jnp.minimum(w, extra)

        def body(i, carry):
            n0 = (start_w + i) * _K
            idx_loads = [
                pltpu.async_copy(idx_hbm.at[pl.ds(j * Nv + n0, _K)],
                                 idx_v.at[j], sem)
                for j in range(Bv)
            ]
            loads = [
                pltpu.async_copy(
                    vals_hbm.at[pl.ds(n0, _K), pl.ds(j * OCv, OCv)],
                    rows_v.at[j],
                    sem,
                )
                for j in range(Bv)
            ]
            for cp in idx_loads + loads:
                cp.wait()
            cps = [
                pltpu.async_copy(
                    rows_v.at[j],
                    out_hbm.at[idx_v.at[j]],
                    sem,
                )
                for j in range(Bv)
            ]
            for cp in cps:
                cp.wait()
            return carry

        lax.fori_loop(0, n_w, body, 0)

        if part:
            idx_p, rows_p = part_bufs

            @pl.when(w == extra)
            def _():
                n0 = nfull * _K
                idx_loads = [
                    pltpu.async_copy(idx_hbm.at[pl.ds(j * Nv + n0, part)],
                                     idx_p.at[j], sem)
                    for j in range(Bv)
                ]
                for cp in idx_loads:
                    cp.wait()
                loads = [
                    pltpu.async_copy(
                        vals_hbm.at[pl.ds(n0, part), pl.ds(j * OCv, OCv)],
                        rows_p.at[j],
                        sem,
                    )
                    for j in range(Bv)
                ]
                for cp in loads:
                    cp.wait()
                cps = [
                    pltpu.async_copy(
                        rows_p.at[j],
                        out_hbm.at[idx_p.at[j]],
                        sem,
                    )
                    for j in range(Bv)
                ]
                for cp in cps:
                    cp.wait()

    return scatter_kernel


def _make_zeros(Mv, OCv):
    _ZB = 2048
    per_w = Mv // _NW
    nfull = per_w // _ZB
    part = per_w - nfull * _ZB

    mesh = plsc.VectorSubcoreMesh(core_axis_name="c", subcore_axis_name="s")

    @functools.partial(
        pl.kernel,
        mesh=mesh,
        out_type=jax.ShapeDtypeStruct((Mv, OCv), jnp.float32),
        compiler_params=pltpu.CompilerParams(use_tc_tiling_on_sc=False,
                                             needs_layout_passes=False),
        scratch_types=[
            pltpu.VMEM((_ZB, OCv), jnp.float32),
            pltpu.SemaphoreType.DMA,
        ],
    )
    def zeros_kernel(out_hbm, zb, sem):
        c = lax.axis_index("c")
        s = lax.axis_index("s")
        w = s * _NC + c
        base = w * per_w
        zero16 = jnp.zeros((OCv,), jnp.float32)

        def zfill(g, carry):
            for t in range(16):
                zb[g * 16 + t] = zero16
            return carry

        lax.fori_loop(0, _ZB // 16, zfill, 0)

        def body(i, carry):
            pltpu.sync_copy(zb, out_hbm.at[pl.ds(base + i * _ZB, _ZB)])
            return carry

        lax.fori_loop(0, nfull, body, 0)
        if part:
            pltpu.sync_copy(zb.at[pl.ds(0, part)],
                            out_hbm.at[pl.ds(base + nfull * _ZB, part)])

    return zeros_kernel


def kernel(shuffled_features, mapping, weights):
    Bv, Nv = mapping.shape
    OCv, _, Cv = weights.shape
    Mv = Bv * Nv
    flat = shuffled_features.reshape(Nv, Cv)
    W2 = jnp.transpose(weights, (2, 1, 0)).reshape(Cv, Bv * OCv)
    vals_wide = _tc_matmul(flat, W2, Nv, Cv, Bv * OCv)

    safe = jnp.maximum(mapping, 0).reshape(-1)

    out_ref = jax.new_ref(_make_zeros(Mv, OCv)())
    _make_scatter(Nv, Bv, OCv)(vals_wide, safe, out_ref)
    out = out_ref[...]

    hit = mapping == 0
    has = jnp.any(hit)
    q = jnp.argmax(hit)
    n0 = q % Nv
    j0 = q // Nv
    row0_src = lax.dynamic_slice(vals_wide, (n0, j0 * OCv), (1, OCv))
    row0 = jnp.where(has, row0_src.reshape(OCv),
                     jnp.zeros((OCv,), jnp.float32))
    return out.at[0].set(row0)
